# baseline (speedup 1.0000x reference)
import functools

import jax
import jax.numpy as jnp
from jax import lax
from jax.experimental import pallas as pl
from jax.experimental.pallas import tpu as pltpu
from jax.experimental.pallas import tpu_sc as plsc

N = 50000
FEAT = 128
MLPD = 64
B = 4096
E = 800000

D = 128
NC = 2
NS = 16
NW = NC * NS
C = 1280
NCHUNK = E // C
K = 128
BP = B + 128
ROWS_PER_TILE = BP // NS
RELS = 3


def _mlp_body(x_ref, w_ref, b_ref, out_ref):
    y = jnp.dot(x_ref[...], w_ref[...], preferred_element_type=jnp.float32,
                precision=lax.Precision.HIGHEST)
    y = jnp.maximum(y + b_ref[...], 0.0)
    rows = y.shape[0]
    ones = jnp.ones((rows, 1), jnp.float32)
    pad = jnp.zeros((rows, D - MLPD - 1), jnp.float32)
    out_ref[...] = jnp.concatenate([y, ones, pad], axis=1)


def _mlp(features, W_mlp, b2d):
    blk = 2000
    grid = N // blk
    return pl.pallas_call(
        _mlp_body,
        grid=(grid,),
        in_specs=[
            pl.BlockSpec((blk, FEAT), lambda i: (i, 0)),
            pl.BlockSpec((FEAT, MLPD), lambda i: (0, 0)),
            pl.BlockSpec((1, MLPD), lambda i: (0, 0)),
        ],
        out_specs=pl.BlockSpec((blk, D), lambda i: (i, 0)),
        out_shape=jax.ShapeDtypeStruct((N, D), jnp.float32),
    )(features, W_mlp, b2d)


ZROWS = ROWS_PER_TILE // 3


def _sc_agg_body(f1aug, inv_hbm, e1, e2, e3, out,
                 inv_v, dst_v, src_v, pend_p, pend_s, pidx, sidx, rowbuf,
                 zbuf, sd, ss, sg, acc):
    c = lax.axis_index("c")
    s = lax.axis_index("s")
    wid = c * NS + s
    edges = [e1, e2, e3]
    base_z = s * ROWS_PER_TILE
    nmy = (NCHUNK - 1 - wid) // NW + 1

    def _zrow(i, _):
        def _zcol(j, __):
            zbuf[i, pl.ds(j * 16, 16)] = jnp.zeros((16,), jnp.float32)
            return 0
        return lax.fori_loop(0, D // 16, _zcol, 0)
    lax.fori_loop(0, ZROWS, _zrow, 0)

    pltpu.sync_copy(inv_hbm, inv_v)

    trash16 = jnp.full((16,), B, jnp.int32)
    zero16 = jnp.zeros((16,), jnp.int32)

    def fire_edges(e, t, ph):
        base = (wid + t * NW) * C
        pltpu.async_copy(e.at[1, pl.ds(base, C)], dst_v[ph], sd[ph])
        pltpu.async_copy(e.at[0, pl.ds(base, C)], src_v[ph], ss[ph])

    def wait_edges(e, ph):
        pltpu.make_async_copy(e.at[1, pl.ds(0, C)], dst_v[ph], sd[ph]).wait()
        pltpu.make_async_copy(e.at[0, pl.ds(0, C)], src_v[ph], ss[ph]).wait()

    def wait_scatter_prev(acc, ph):
        oph = 1 - ph
        pltpu.make_async_copy(f1aug.at[sidx[oph]], rowbuf[oph],
                              sg[oph]).wait()
        pltpu.sync_copy(rowbuf[oph], acc.at[pidx[oph]], add=True)

    def filter_chunk(acc, ph):
        dv, sv, pp, ps = dst_v[ph], src_v[ph], pend_p[ph], pend_s[ph]

        def vbody(j, cnt):
            dvec = dv[pl.ds(j * 16, 16)]
            word = plsc.load_gather(inv_v, [lax.shift_right_logical(dvec, 1)])
            sh = lax.shift_left(dvec & 1, 4)
            v = lax.shift_right_logical(word, sh) & 0xFFFF
            p = v - 1
            m = v > 0
            svec = sv[pl.ds(j * 16, 16)]
            plsc.store_compressed(pp.at[pl.ds(cnt, 16)], p, mask=m)
            plsc.store_compressed(ps.at[pl.ds(cnt, 16)], svec, mask=m)
            return cnt + jnp.sum(m.astype(jnp.int32))
        cnt = lax.fori_loop(0, C // 16, vbody, 0)

        for j in range(K // 16):
            pp[pl.ds(cnt + j * 16, 16)] = trash16
            ps[pl.ds(cnt + j * 16, 16)] = zero16

        nflush = (cnt + K - 1) // K

        def fbody(f, __):
            off = f * K
            def cpy(j, ___):
                pidx[ph][pl.ds(j * 16, 16)] = pp[pl.ds(off + j * 16, 16)]
                sidx[ph][pl.ds(j * 16, 16)] = ps[pl.ds(off + j * 16, 16)]
                return 0
            lax.fori_loop(0, K // 16, cpy, 0)
            pltpu.async_copy(f1aug.at[sidx[ph]], rowbuf[ph], sg[ph]).wait()
            pltpu.sync_copy(rowbuf[ph], acc.at[pidx[ph]], add=True)
            return 0
        lax.fori_loop(1, nflush, fbody, 0)
        return cnt

    def fire_flush0(acc, ph):
        def cpy(j, ___):
            pidx[ph][pl.ds(j * 16, 16)] = pend_p[ph][pl.ds(j * 16, 16)]
            sidx[ph][pl.ds(j * 16, 16)] = pend_s[ph][pl.ds(j * 16, 16)]
            return 0
        lax.fori_loop(0, K // 16, cpy, 0)
        pltpu.async_copy(f1aug.at[sidx[ph]], rowbuf[ph], sg[ph])

    for r in range(RELS):
        e = edges[r]

        for z in range(3):
            pltpu.sync_copy(zbuf, acc.at[pl.ds(base_z + z * ZROWS, ZROWS)])
        plsc.subcore_barrier()

        fire_edges(e, 0, 0)

        def chunk_step(t, ph, e=e, acc=acc):
            wait_edges(e, ph)

            @pl.when(t + 1 < nmy)
            def _():
                fire_edges(e, t + 1, 1 - ph)
            filter_chunk(acc, ph)

            @pl.when(t > 0)
            def _():
                wait_scatter_prev(acc, ph)
            fire_flush0(acc, ph)

        def pair_body(u, _, e=e, acc=acc):
            chunk_step(2 * u, 0)

            @pl.when(2 * u + 1 < nmy)
            def _():
                chunk_step(2 * u + 1, 1)
            return 0

        lax.fori_loop(0, (nmy + 1) // 2, pair_body, 0)

        @pl.when(nmy % 2 == 1)
        def _():
            wait_scatter_prev(acc, 1)

        @pl.when(nmy % 2 == 0)
        def _():
            wait_scatter_prev(acc, 0)

        plsc.subcore_barrier()
        pltpu.sync_copy(
            acc.at[pl.ds(base_z, ROWS_PER_TILE)],
            out.at[c * RELS + r, pl.ds(base_z, ROWS_PER_TILE)])
        plsc.subcore_barrier()


def _sc_agg(f1aug, inv, e1, e2, e3):
    mesh = plsc.VectorSubcoreMesh(core_axis_name="c", subcore_axis_name="s")
    fn = functools.partial(
        pl.kernel,
        out_type=jax.ShapeDtypeStruct((NC * RELS, BP, D), jnp.float32),
        mesh=mesh,
        compiler_params=pltpu.CompilerParams(needs_layout_passes=False),
        scratch_types=[
            pltpu.VMEM((N // 2,), jnp.int32),
            (pltpu.VMEM((C,), jnp.int32), pltpu.VMEM((C,), jnp.int32)),
            (pltpu.VMEM((C,), jnp.int32), pltpu.VMEM((C,), jnp.int32)),
            (pltpu.VMEM((C + K + 16,), jnp.int32),
             pltpu.VMEM((C + K + 16,), jnp.int32)),
            (pltpu.VMEM((C + K + 16,), jnp.int32),
             pltpu.VMEM((C + K + 16,), jnp.int32)),
            (pltpu.VMEM((K,), jnp.int32), pltpu.VMEM((K,), jnp.int32)),
            (pltpu.VMEM((K,), jnp.int32), pltpu.VMEM((K,), jnp.int32)),
            (pltpu.VMEM((K, D), jnp.float32), pltpu.VMEM((K, D), jnp.float32)),
            pltpu.VMEM((ZROWS, D), jnp.float32),
            (pltpu.SemaphoreType.DMA, pltpu.SemaphoreType.DMA),
            (pltpu.SemaphoreType.DMA, pltpu.SemaphoreType.DMA),
            (pltpu.SemaphoreType.DMA, pltpu.SemaphoreType.DMA),
            pltpu.VMEM_SHARED((BP, D), jnp.float32),
        ],
    )(_sc_agg_body)
    return fn(f1aug, inv, e1, e2, e3)


def _sc_batch_body(parts, pb, nodes, f1aug, br_out, f1b_out,
                   idxv, nidx, buf, sem):
    c = lax.axis_index("c")
    s = lax.axis_index("s")
    wid = c * NS + s
    nb = B // NW
    base = wid * nb

    pltpu.sync_copy(pb.at[pl.ds(base, nb)], idxv)
    pltpu.sync_copy(nodes.at[pl.ds(base, nb)], nidx)

    pltpu.async_copy(f1aug.at[nidx], buf, sem).wait()
    pltpu.sync_copy(buf, f1b_out.at[pl.ds(base, nb)])

    for r in range(RELS):
        pltpu.async_copy(parts.at[r].at[idxv], buf, sem).wait()
        pltpu.async_copy(parts.at[RELS + r].at[idxv], buf, sem, add=True).wait()
        pltpu.sync_copy(buf, br_out.at[r, pl.ds(base, nb)])


def _sc_batch(parts, pb, nodes, f1aug):
    mesh = plsc.VectorSubcoreMesh(core_axis_name="c", subcore_axis_name="s")
    fn = functools.partial(
        pl.kernel,
        out_type=(jax.ShapeDtypeStruct((RELS, B, D), jnp.float32),
                  jax.ShapeDtypeStruct((B, D), jnp.float32)),
        mesh=mesh,
        compiler_params=pltpu.CompilerParams(needs_layout_passes=False),
        scratch_types=[
            pltpu.VMEM((B // NW,), jnp.int32),
            pltpu.VMEM((B // NW,), jnp.int32),
            pltpu.VMEM((B // NW, D), jnp.float32),
            pltpu.SemaphoreType.DMA,
        ],
    )(_sc_batch_body)
    return fn(parts, pb, nodes, f1aug)


def _head_body(f1b_ref, br_ref, lab_ref, w1, w2, w3, ws1, ws2, ws3, wt,
               comb_ref, loss_ref):
    i = pl.program_id(0)
    f1 = f1b_ref[:, :MLPD]
    lab = lab_ref[...]
    hs = [f1]
    loss = jnp.zeros((1, 1), jnp.float32)
    for r, (w, ws) in enumerate(((w1, ws1), (w2, ws2), (w3, ws3))):
        row = br_ref[r]
        ssum = row[:, :MLPD]
        deg = row[:, MLPD:MLPD + 1]
        neigh = ssum / jnp.maximum(deg, 1.0)
        cat = jnp.concatenate([f1, neigh], axis=1)
        h = jnp.maximum(
            jnp.dot(cat, w[...], preferred_element_type=jnp.float32,
                    precision=lax.Precision.HIGHEST), 0.0)
        hs.append(h)
        logits = jnp.dot(h, ws[...], preferred_element_type=jnp.float32,
                         precision=lax.Precision.HIGHEST)
        l0 = logits[:, 0:1]
        l1 = logits[:, 1:2]
        m = jnp.maximum(l0, l1)
        lse = m + jnp.log(jnp.exp(l0 - m) + jnp.exp(l1 - m))
        ll = jnp.where(lab == 0, l0, l1)
        loss = loss + jnp.sum(lse - ll, keepdims=True).reshape(1, 1) / B
    cat2 = jnp.concatenate(hs, axis=1)
    comb_ref[...] = jnp.maximum(
        jnp.dot(cat2, wt[...], preferred_element_type=jnp.float32,
                precision=lax.Precision.HIGHEST), 0.0)

    @pl.when(i == 0)
    def _():
        loss_ref[...] = jnp.zeros((1, 1), jnp.float32)
    loss_ref[...] += loss


def _tc_head(f1b, br, lab2d, W1, W2, W3, Ws1, Ws2, Ws3, weight):
    blk = 1024
    grid = B // blk
    return pl.pallas_call(
        _head_body,
        grid=(grid,),
        in_specs=[
            pl.BlockSpec((blk, D), lambda i: (i, 0)),
            pl.BlockSpec((RELS, blk, D), lambda i: (0, i, 0)),
            pl.BlockSpec((blk, 1), lambda i: (i, 0)),
            pl.BlockSpec((2 * MLPD, MLPD), lambda i: (0, 0)),
            pl.BlockSpec((2 * MLPD, MLPD), lambda i: (0, 0)),
            pl.BlockSpec((2 * MLPD, MLPD), lambda i: (0, 0)),
            pl.BlockSpec((MLPD, 2), lambda i: (0, 0)),
            pl.BlockSpec((MLPD, 2), lambda i: (0, 0)),
            pl.BlockSpec((MLPD, 2), lambda i: (0, 0)),
            pl.BlockSpec((MLPD + 3 * MLPD, MLPD), lambda i: (0, 0)),
        ],
        out_specs=(pl.BlockSpec((blk, MLPD), lambda i: (i, 0)),
                   pl.BlockSpec((1, 1), lambda i: (0, 0))),
        out_shape=(jax.ShapeDtypeStruct((B, MLPD), jnp.float32),
                   jax.ShapeDtypeStruct((1, 1), jnp.float32)),
    )(f1b, br, lab2d, W1, W2, W3, Ws1, Ws2, Ws3, weight)


def kernel(features, nodes, labels, edge_index1, edge_index2, edge_index3,
           W_mlp, b_mlp, W1, W2, W3, Ws1, Ws2, Ws3, weight):
    nodes = nodes.astype(jnp.int32)
    f1aug = _mlp(features, W_mlp, b_mlp.reshape(1, MLPD))
    inv = jnp.full((N,), -1, jnp.int32).at[nodes].set(
        jnp.arange(B, dtype=jnp.int32))
    pb = inv[nodes]
    invp1 = inv + 1
    inv_packed = invp1[0::2] | (invp1[1::2] << 16)
    parts = _sc_agg(f1aug, inv_packed,
                    edge_index1.astype(jnp.int32),
                    edge_index2.astype(jnp.int32),
                    edge_index3.astype(jnp.int32))
    br, f1b = _sc_batch(parts, pb, nodes, f1aug)
    comb, loss = _tc_head(f1b, br, labels.reshape(B, 1).astype(jnp.int32),
                          W1, W2, W3, Ws1, Ws2, Ws3, weight)
    return comb.T, f1b[:, :MLPD].T, loss.reshape(())


# speedup vs baseline: 4.0507x; 4.0507x over previous
import functools

import jax
import jax.numpy as jnp
from jax import lax
from jax.experimental import pallas as pl
from jax.experimental.pallas import tpu as pltpu
from jax.experimental.pallas import tpu_sc as plsc

N = 50000
FEAT = 128
MLPD = 64
B = 4096
E = 800000

D = 128
NC = 2
NS = 16
NW = NC * NS
C = 1280
NCHUNK = E // C
K = 128
BP = B + 128
ROWS_PER_TILE = BP // NS
RELS = 3


def _mlp_body(x_ref, w_ref, b_ref, out_ref):
    y = jnp.dot(x_ref[...], w_ref[...], preferred_element_type=jnp.float32,
                precision=lax.Precision.HIGHEST)
    y = jnp.maximum(y + b_ref[...], 0.0)
    rows = y.shape[0]
    ones = jnp.ones((rows, 1), jnp.float32)
    pad = jnp.zeros((rows, D - MLPD - 1), jnp.float32)
    out_ref[...] = jnp.concatenate([y, ones, pad], axis=1)


def _mlp(features, W_mlp, b2d):
    blk = 2000
    grid = N // blk
    return pl.pallas_call(
        _mlp_body,
        grid=(grid,),
        in_specs=[
            pl.BlockSpec((blk, FEAT), lambda i: (i, 0)),
            pl.BlockSpec((FEAT, MLPD), lambda i: (0, 0)),
            pl.BlockSpec((1, MLPD), lambda i: (0, 0)),
        ],
        out_specs=pl.BlockSpec((blk, D), lambda i: (i, 0)),
        out_shape=jax.ShapeDtypeStruct((N, D), jnp.float32),
    )(features, W_mlp, b2d)


ZROWS = ROWS_PER_TILE // 3


def _sc_agg_body(f1aug, inv_hbm, e1, e2, e3, out,
                 inv_v, dst_v, src_v, pend_p, pend_s, pidx, sidx, rowbuf,
                 zbuf, sd, ss, sg, acc):
    c = lax.axis_index("c")
    s = lax.axis_index("s")
    wid = c * NS + s
    edges = [e1, e2, e3]
    base_z = s * ROWS_PER_TILE
    nmy = (NCHUNK - 1 - wid) // NW + 1

    def _zrow(i, _):
        def _zcol(j, __):
            zbuf[i, pl.ds(j * 16, 16)] = jnp.zeros((16,), jnp.float32)
            return 0
        return lax.fori_loop(0, D // 16, _zcol, 0)
    lax.fori_loop(0, ZROWS, _zrow, 0)

    pltpu.sync_copy(inv_hbm, inv_v)

    trash16 = jnp.full((16,), B, jnp.int32)
    zero16 = jnp.zeros((16,), jnp.int32)

    def fire_edges(e, t, ph):
        base = (wid + t * NW) * C
        pltpu.async_copy(e.at[1, pl.ds(base, C)], dst_v[ph], sd[ph])
        pltpu.async_copy(e.at[0, pl.ds(base, C)], src_v[ph], ss[ph])

    def wait_edges(e, ph):
        pltpu.make_async_copy(e.at[1, pl.ds(0, C)], dst_v[ph], sd[ph]).wait()
        pltpu.make_async_copy(e.at[0, pl.ds(0, C)], src_v[ph], ss[ph]).wait()

    def flush_block(acc, off):
        def cpy(j, ___):
            pidx[pl.ds(j * 16, 16)] = pend_p[pl.ds(off + j * 16, 16)]
            sidx[pl.ds(j * 16, 16)] = pend_s[pl.ds(off + j * 16, 16)]
            return 0
        lax.fori_loop(0, K // 16, cpy, 0)
        pltpu.async_copy(f1aug.at[sidx], rowbuf, sg).wait()
        pltpu.sync_copy(rowbuf, acc.at[pidx], add=True)

    def filter_chunk(acc, ph, rem):
        dv, sv = dst_v[ph], src_v[ph]

        def vbody(j, cnt):
            dvec = dv[pl.ds(j * 16, 16)]
            word = plsc.load_gather(inv_v, [lax.shift_right_logical(dvec, 1)])
            sh = lax.shift_left(dvec & 1, 4)
            v = lax.shift_right_logical(word, sh) & 0xFFFF
            p = v - 1
            m = v > 0
            svec = sv[pl.ds(j * 16, 16)]
            plsc.store_compressed(pend_p.at[pl.ds(cnt, 16)], p, mask=m)
            plsc.store_compressed(pend_s.at[pl.ds(cnt, 16)], svec, mask=m)
            return cnt + jnp.sum(m.astype(jnp.int32))
        cnt = lax.fori_loop(0, C // 16, vbody, rem)

        nflush = cnt // K

        def fbody(f, __):
            flush_block(acc, f * K)
            return 0
        lax.fori_loop(0, nflush, fbody, 0)

        @pl.when(nflush > 0)
        def _():
            def mv(j, __):
                pend_p[pl.ds(j * 16, 16)] = \
                    pend_p[pl.ds(nflush * K + j * 16, 16)]
                pend_s[pl.ds(j * 16, 16)] = \
                    pend_s[pl.ds(nflush * K + j * 16, 16)]
                return 0
            lax.fori_loop(0, K // 16, mv, 0)
        return cnt - nflush * K

    def flush_tail(acc, rem):
        def pad(j, __):
            pend_p[pl.ds(rem + j * 16, 16)] = trash16
            pend_s[pl.ds(rem + j * 16, 16)] = zero16
            return 0
        lax.fori_loop(0, K // 16, pad, 0)

        @pl.when(rem > 0)
        def _():
            flush_block(acc, 0)

    for r in range(RELS):
        e = edges[r]

        for z in range(3):
            pltpu.sync_copy(zbuf, acc.at[pl.ds(base_z + z * ZROWS, ZROWS)])
        plsc.subcore_barrier()

        fire_edges(e, 0, 0)

        def chunk_step(t, ph, rem, e=e, acc=acc):
            wait_edges(e, ph)

            @pl.when(t + 1 < nmy)
            def _():
                fire_edges(e, t + 1, 1 - ph)
            return filter_chunk(acc, ph, rem)

        def pair_body(u, rem, e=e, acc=acc):
            rem = chunk_step(2 * u, 0, rem)
            return lax.cond(2 * u + 1 < nmy,
                            lambda rm: chunk_step(2 * u + 1, 1, rm),
                            lambda rm: rm, rem)

        rem = lax.fori_loop(0, (nmy + 1) // 2, pair_body, 0)
        flush_tail(acc, rem)

        plsc.subcore_barrier()
        pltpu.sync_copy(
            acc.at[pl.ds(base_z, ROWS_PER_TILE)],
            out.at[c * RELS + r, pl.ds(base_z, ROWS_PER_TILE)])
        plsc.subcore_barrier()


def _sc_agg(f1aug, inv, e1, e2, e3):
    mesh = plsc.VectorSubcoreMesh(core_axis_name="c", subcore_axis_name="s")
    fn = functools.partial(
        pl.kernel,
        out_type=jax.ShapeDtypeStruct((NC * RELS, BP, D), jnp.float32),
        mesh=mesh,
        compiler_params=pltpu.CompilerParams(needs_layout_passes=False),
        scratch_types=[
            pltpu.VMEM((N // 2,), jnp.int32),
            (pltpu.VMEM((C,), jnp.int32), pltpu.VMEM((C,), jnp.int32)),
            (pltpu.VMEM((C,), jnp.int32), pltpu.VMEM((C,), jnp.int32)),
            pltpu.VMEM((C + K + 16,), jnp.int32),
            pltpu.VMEM((C + K + 16,), jnp.int32),
            pltpu.VMEM((K,), jnp.int32),
            pltpu.VMEM((K,), jnp.int32),
            pltpu.VMEM((K, D), jnp.float32),
            pltpu.VMEM((ZROWS, D), jnp.float32),
            (pltpu.SemaphoreType.DMA, pltpu.SemaphoreType.DMA),
            (pltpu.SemaphoreType.DMA, pltpu.SemaphoreType.DMA),
            pltpu.SemaphoreType.DMA,
            pltpu.VMEM_SHARED((BP, D), jnp.float32),
        ],
    )(_sc_agg_body)
    return fn(f1aug, inv, e1, e2, e3)


def _sc_batch_body(parts, pb, nodes, f1aug, br_out, f1b_out,
                   idxv, nidx, buf, sem):
    c = lax.axis_index("c")
    s = lax.axis_index("s")
    wid = c * NS + s
    nb = B // NW
    base = wid * nb

    pltpu.sync_copy(pb.at[pl.ds(base, nb)], idxv)
    pltpu.sync_copy(nodes.at[pl.ds(base, nb)], nidx)

    pltpu.async_copy(f1aug.at[nidx], buf, sem).wait()
    pltpu.sync_copy(buf, f1b_out.at[pl.ds(base, nb)])

    for r in range(RELS):
        pltpu.async_copy(parts.at[r].at[idxv], buf, sem).wait()
        pltpu.async_copy(parts.at[RELS + r].at[idxv], buf, sem, add=True).wait()
        pltpu.sync_copy(buf, br_out.at[r, pl.ds(base, nb)])


def _sc_batch(parts, pb, nodes, f1aug):
    mesh = plsc.VectorSubcoreMesh(core_axis_name="c", subcore_axis_name="s")
    fn = functools.partial(
        pl.kernel,
        out_type=(jax.ShapeDtypeStruct((RELS, B, D), jnp.float32),
                  jax.ShapeDtypeStruct((B, D), jnp.float32)),
        mesh=mesh,
        compiler_params=pltpu.CompilerParams(needs_layout_passes=False),
        scratch_types=[
            pltpu.VMEM((B // NW,), jnp.int32),
            pltpu.VMEM((B // NW,), jnp.int32),
            pltpu.VMEM((B // NW, D), jnp.float32),
            pltpu.SemaphoreType.DMA,
        ],
    )(_sc_batch_body)
    return fn(parts, pb, nodes, f1aug)


def _head_body(f1b_ref, br_ref, lab_ref, w1, w2, w3, ws1, ws2, ws3, wt,
               comb_ref, loss_ref):
    i = pl.program_id(0)
    f1 = f1b_ref[:, :MLPD]
    lab = lab_ref[...]
    hs = [f1]
    loss = jnp.zeros((1, 1), jnp.float32)
    for r, (w, ws) in enumerate(((w1, ws1), (w2, ws2), (w3, ws3))):
        row = br_ref[r]
        ssum = row[:, :MLPD]
        deg = row[:, MLPD:MLPD + 1]
        neigh = ssum / jnp.maximum(deg, 1.0)
        cat = jnp.concatenate([f1, neigh], axis=1)
        h = jnp.maximum(
            jnp.dot(cat, w[...], preferred_element_type=jnp.float32,
                    precision=lax.Precision.HIGHEST), 0.0)
        hs.append(h)
        logits = jnp.dot(h, ws[...], preferred_element_type=jnp.float32,
                         precision=lax.Precision.HIGHEST)
        l0 = logits[:, 0:1]
        l1 = logits[:, 1:2]
        m = jnp.maximum(l0, l1)
        lse = m + jnp.log(jnp.exp(l0 - m) + jnp.exp(l1 - m))
        ll = jnp.where(lab == 0, l0, l1)
        loss = loss + jnp.sum(lse - ll, keepdims=True).reshape(1, 1) / B
    cat2 = jnp.concatenate(hs, axis=1)
    comb_ref[...] = jnp.maximum(
        jnp.dot(cat2, wt[...], preferred_element_type=jnp.float32,
                precision=lax.Precision.HIGHEST), 0.0)

    @pl.when(i == 0)
    def _():
        loss_ref[...] = jnp.zeros((1, 1), jnp.float32)
    loss_ref[...] += loss


def _tc_head(f1b, br, lab2d, W1, W2, W3, Ws1, Ws2, Ws3, weight):
    blk = 1024
    grid = B // blk
    return pl.pallas_call(
        _head_body,
        grid=(grid,),
        in_specs=[
            pl.BlockSpec((blk, D), lambda i: (i, 0)),
            pl.BlockSpec((RELS, blk, D), lambda i: (0, i, 0)),
            pl.BlockSpec((blk, 1), lambda i: (i, 0)),
            pl.BlockSpec((2 * MLPD, MLPD), lambda i: (0, 0)),
            pl.BlockSpec((2 * MLPD, MLPD), lambda i: (0, 0)),
            pl.BlockSpec((2 * MLPD, MLPD), lambda i: (0, 0)),
            pl.BlockSpec((MLPD, 2), lambda i: (0, 0)),
            pl.BlockSpec((MLPD, 2), lambda i: (0, 0)),
            pl.BlockSpec((MLPD, 2), lambda i: (0, 0)),
            pl.BlockSpec((MLPD + 3 * MLPD, MLPD), lambda i: (0, 0)),
        ],
        out_specs=(pl.BlockSpec((blk, MLPD), lambda i: (i, 0)),
                   pl.BlockSpec((1, 1), lambda i: (0, 0))),
        out_shape=(jax.ShapeDtypeStruct((B, MLPD), jnp.float32),
                   jax.ShapeDtypeStruct((1, 1), jnp.float32)),
    )(f1b, br, lab2d, W1, W2, W3, Ws1, Ws2, Ws3, weight)


def kernel(features, nodes, labels, edge_index1, edge_index2, edge_index3,
           W_mlp, b_mlp, W1, W2, W3, Ws1, Ws2, Ws3, weight):
    nodes = nodes.astype(jnp.int32)
    f1aug = _mlp(features, W_mlp, b_mlp.reshape(1, MLPD))
    inv = jnp.full((N,), -1, jnp.int32).at[nodes].set(
        jnp.arange(B, dtype=jnp.int32))
    pb = inv[nodes]
    invp1 = inv + 1
    inv_packed = invp1[0::2] | (invp1[1::2] << 16)
    parts = _sc_agg(f1aug, inv_packed,
                    edge_index1.astype(jnp.int32),
                    edge_index2.astype(jnp.int32),
                    edge_index3.astype(jnp.int32))
    br, f1b = _sc_batch(parts, pb, nodes, f1aug)
    comb, loss = _tc_head(f1b, br, labels.reshape(B, 1).astype(jnp.int32),
                          W1, W2, W3, Ws1, Ws2, Ws3, weight)
    return comb.T, f1b[:, :MLPD].T, loss.reshape(())


# speedup vs baseline: 4.4275x; 1.0930x over previous
import functools

import jax
import jax.numpy as jnp
from jax import lax
from jax.experimental import pallas as pl
from jax.experimental.pallas import tpu as pltpu
from jax.experimental.pallas import tpu_sc as plsc

N = 50000
FEAT = 128
MLPD = 64
B = 4096
E = 800000

D = 128
NC = 2
NS = 16
NW = NC * NS
C = 1280
NCHUNK = E // C
K = 128
BP = B + 128
ROWS_PER_TILE = BP // NS
RELS = 3


def _mlp_body(x_ref, w_ref, b_ref, out_ref):
    y = jnp.dot(x_ref[...], w_ref[...], preferred_element_type=jnp.float32,
                precision=lax.Precision.HIGHEST)
    y = jnp.maximum(y + b_ref[...], 0.0)
    rows = y.shape[0]
    ones = jnp.ones((rows, 1), jnp.float32)
    pad = jnp.zeros((rows, D - MLPD - 1), jnp.float32)
    out_ref[...] = jnp.concatenate([y, ones, pad], axis=1)


def _mlp(features, W_mlp, b2d):
    blk = 2000
    grid = N // blk
    return pl.pallas_call(
        _mlp_body,
        grid=(grid,),
        in_specs=[
            pl.BlockSpec((blk, FEAT), lambda i: (i, 0)),
            pl.BlockSpec((FEAT, MLPD), lambda i: (0, 0)),
            pl.BlockSpec((1, MLPD), lambda i: (0, 0)),
        ],
        out_specs=pl.BlockSpec((blk, D), lambda i: (i, 0)),
        out_shape=jax.ShapeDtypeStruct((N, D), jnp.float32),
    )(features, W_mlp, b2d)


ZROWS = ROWS_PER_TILE // 3


def _sc_agg_body(f1aug, inv_hbm, e1, e2, e3, out,
                 inv_v, dst_v, src_v, pend_p, pend_s, pidx, sidx, rowbuf,
                 zbuf, sd, ss, sg, acc):
    c = lax.axis_index("c")
    s = lax.axis_index("s")
    wid = c * NS + s
    edges = [e1, e2, e3]
    base_z = s * ROWS_PER_TILE
    nmy = (NCHUNK - 1 - wid) // NW + 1

    def _zrow(i, _):
        def _zcol(j, __):
            zbuf[i, pl.ds(j * 16, 16)] = jnp.zeros((16,), jnp.float32)
            return 0
        return lax.fori_loop(0, D // 16, _zcol, 0)
    lax.fori_loop(0, ZROWS, _zrow, 0)

    pltpu.sync_copy(inv_hbm, inv_v)

    trash16 = jnp.full((16,), B, jnp.int32)
    zero16 = jnp.zeros((16,), jnp.int32)

    def fire_edges(e, t, ph):
        base = (wid + t * NW) * C
        pltpu.async_copy(e.at[1, pl.ds(base, C)], dst_v[ph], sd[ph])
        pltpu.async_copy(e.at[0, pl.ds(base, C)], src_v[ph], ss[ph])

    def wait_edges(e, ph):
        pltpu.make_async_copy(e.at[1, pl.ds(0, C)], dst_v[ph], sd[ph]).wait()
        pltpu.make_async_copy(e.at[0, pl.ds(0, C)], src_v[ph], ss[ph]).wait()

    def drain_flush(acc, f):
        ph = f & 1
        pltpu.make_async_copy(f1aug.at[sidx.at[ph]], rowbuf.at[ph],
                              sg.at[ph]).wait()
        pltpu.sync_copy(rowbuf.at[ph], acc.at[pidx.at[ph]], add=True)

    def fire_flush(f, off):
        ph = f & 1

        def cpy(j, ___):
            pidx[ph, pl.ds(j * 16, 16)] = pend_p[pl.ds(off + j * 16, 16)]
            sidx[ph, pl.ds(j * 16, 16)] = pend_s[pl.ds(off + j * 16, 16)]
            return 0
        lax.fori_loop(0, K // 16, cpy, 0)
        pltpu.async_copy(f1aug.at[sidx.at[ph]], rowbuf.at[ph], sg.at[ph])

    def filter_chunk(acc, ph, rem, fc):
        dv, sv = dst_v[ph], src_v[ph]

        def vbody(j, cnt):
            dvec = dv[pl.ds(j * 16, 16)]
            word = plsc.load_gather(inv_v, [lax.shift_right_logical(dvec, 1)])
            sh = lax.shift_left(dvec & 1, 4)
            v = lax.shift_right_logical(word, sh) & 0xFFFF
            p = v - 1
            m = v > 0
            svec = sv[pl.ds(j * 16, 16)]
            plsc.store_compressed(pend_p.at[pl.ds(cnt, 16)], p, mask=m)
            plsc.store_compressed(pend_s.at[pl.ds(cnt, 16)], svec, mask=m)
            return cnt + jnp.sum(m.astype(jnp.int32))
        cnt = lax.fori_loop(0, C // 16, vbody, rem)

        nflush = cnt // K

        def fbody(f, fc2):
            @pl.when(fc2 > 0)
            def _():
                drain_flush(acc, fc2 - 1)
            fire_flush(fc2, f * K)
            return fc2 + 1
        fc = lax.fori_loop(0, nflush, fbody, fc)

        @pl.when(nflush > 0)
        def _():
            def mv(j, __):
                pend_p[pl.ds(j * 16, 16)] = \
                    pend_p[pl.ds(nflush * K + j * 16, 16)]
                pend_s[pl.ds(j * 16, 16)] = \
                    pend_s[pl.ds(nflush * K + j * 16, 16)]
                return 0
            lax.fori_loop(0, K // 16, mv, 0)
        return cnt - nflush * K, fc

    def flush_tail(acc, rem, fc):
        def pad(j, __):
            pend_p[pl.ds(rem + j * 16, 16)] = trash16
            pend_s[pl.ds(rem + j * 16, 16)] = zero16
            return 0
        lax.fori_loop(0, K // 16, pad, 0)

        def last_fire(fc2):
            @pl.when(fc2 > 0)
            def _():
                drain_flush(acc, fc2 - 1)
            fire_flush(fc2, 0)
            return fc2 + 1
        fc = lax.cond(rem > 0, last_fire, lambda fc2: fc2, fc)

        @pl.when(fc > 0)
        def _():
            drain_flush(acc, fc - 1)

    for r in range(RELS):
        e = edges[r]

        for z in range(3):
            pltpu.sync_copy(zbuf, acc.at[pl.ds(base_z + z * ZROWS, ZROWS)])
        plsc.subcore_barrier()

        fire_edges(e, 0, 0)

        def chunk_step(t, ph, st, e=e, acc=acc):
            wait_edges(e, ph)

            @pl.when(t + 1 < nmy)
            def _():
                fire_edges(e, t + 1, 1 - ph)
            return filter_chunk(acc, ph, st[0], st[1])

        def pair_body(u, st, e=e, acc=acc):
            st = chunk_step(2 * u, 0, st)
            return lax.cond(2 * u + 1 < nmy,
                            lambda s: chunk_step(2 * u + 1, 1, s),
                            lambda s: s, st)

        rem, fc = lax.fori_loop(0, (nmy + 1) // 2, pair_body, (0, 0))
        flush_tail(acc, rem, fc)

        plsc.subcore_barrier()
        pltpu.sync_copy(
            acc.at[pl.ds(base_z, ROWS_PER_TILE)],
            out.at[c * RELS + r, pl.ds(base_z, ROWS_PER_TILE)])
        plsc.subcore_barrier()


def _sc_agg(f1aug, inv, e1, e2, e3):
    mesh = plsc.VectorSubcoreMesh(core_axis_name="c", subcore_axis_name="s")
    fn = functools.partial(
        pl.kernel,
        out_type=jax.ShapeDtypeStruct((NC * RELS, BP, D), jnp.float32),
        mesh=mesh,
        compiler_params=pltpu.CompilerParams(needs_layout_passes=False),
        scratch_types=[
            pltpu.VMEM((N // 2,), jnp.int32),
            (pltpu.VMEM((C,), jnp.int32), pltpu.VMEM((C,), jnp.int32)),
            (pltpu.VMEM((C,), jnp.int32), pltpu.VMEM((C,), jnp.int32)),
            pltpu.VMEM((C + K + 16,), jnp.int32),
            pltpu.VMEM((C + K + 16,), jnp.int32),
            pltpu.VMEM((2, K), jnp.int32),
            pltpu.VMEM((2, K), jnp.int32),
            pltpu.VMEM((2, K, D), jnp.float32),
            pltpu.VMEM((ZROWS, D), jnp.float32),
            (pltpu.SemaphoreType.DMA, pltpu.SemaphoreType.DMA),
            (pltpu.SemaphoreType.DMA, pltpu.SemaphoreType.DMA),
            pltpu.SemaphoreType.DMA((2,)),
            pltpu.VMEM_SHARED((BP, D), jnp.float32),
        ],
    )(_sc_agg_body)
    return fn(f1aug, inv, e1, e2, e3)


def _sc_batch_body(parts, pb, nodes, f1aug, br_out, f1b_out,
                   idxv, nidx, buf, sem):
    c = lax.axis_index("c")
    s = lax.axis_index("s")
    wid = c * NS + s
    nb = B // NW
    base = wid * nb

    pltpu.sync_copy(pb.at[pl.ds(base, nb)], idxv)
    pltpu.sync_copy(nodes.at[pl.ds(base, nb)], nidx)

    pltpu.async_copy(f1aug.at[nidx], buf, sem).wait()
    pltpu.sync_copy(buf, f1b_out.at[pl.ds(base, nb)])

    for r in range(RELS):
        pltpu.async_copy(parts.at[r].at[idxv], buf, sem).wait()
        pltpu.async_copy(parts.at[RELS + r].at[idxv], buf, sem, add=True).wait()
        pltpu.sync_copy(buf, br_out.at[r, pl.ds(base, nb)])


def _sc_batch(parts, pb, nodes, f1aug):
    mesh = plsc.VectorSubcoreMesh(core_axis_name="c", subcore_axis_name="s")
    fn = functools.partial(
        pl.kernel,
        out_type=(jax.ShapeDtypeStruct((RELS, B, D), jnp.float32),
                  jax.ShapeDtypeStruct((B, D), jnp.float32)),
        mesh=mesh,
        compiler_params=pltpu.CompilerParams(needs_layout_passes=False),
        scratch_types=[
            pltpu.VMEM((B // NW,), jnp.int32),
            pltpu.VMEM((B // NW,), jnp.int32),
            pltpu.VMEM((B // NW, D), jnp.float32),
            pltpu.SemaphoreType.DMA,
        ],
    )(_sc_batch_body)
    return fn(parts, pb, nodes, f1aug)


def _head_body(f1b_ref, br_ref, lab_ref, w1, w2, w3, ws1, ws2, ws3, wt,
               comb_ref, loss_ref):
    i = pl.program_id(0)
    f1 = f1b_ref[:, :MLPD]
    lab = lab_ref[...]
    hs = [f1]
    loss = jnp.zeros((1, 1), jnp.float32)
    for r, (w, ws) in enumerate(((w1, ws1), (w2, ws2), (w3, ws3))):
        row = br_ref[r]
        ssum = row[:, :MLPD]
        deg = row[:, MLPD:MLPD + 1]
        neigh = ssum / jnp.maximum(deg, 1.0)
        cat = jnp.concatenate([f1, neigh], axis=1)
        h = jnp.maximum(
            jnp.dot(cat, w[...], preferred_element_type=jnp.float32,
                    precision=lax.Precision.HIGHEST), 0.0)
        hs.append(h)
        logits = jnp.dot(h, ws[...], preferred_element_type=jnp.float32,
                         precision=lax.Precision.HIGHEST)
        l0 = logits[:, 0:1]
        l1 = logits[:, 1:2]
        m = jnp.maximum(l0, l1)
        lse = m + jnp.log(jnp.exp(l0 - m) + jnp.exp(l1 - m))
        ll = jnp.where(lab == 0, l0, l1)
        loss = loss + jnp.sum(lse - ll, keepdims=True).reshape(1, 1) / B
    cat2 = jnp.concatenate(hs, axis=1)
    comb_ref[...] = jnp.maximum(
        jnp.dot(cat2, wt[...], preferred_element_type=jnp.float32,
                precision=lax.Precision.HIGHEST), 0.0)

    @pl.when(i == 0)
    def _():
        loss_ref[...] = jnp.zeros((1, 1), jnp.float32)
    loss_ref[...] += loss


def _tc_head(f1b, br, lab2d, W1, W2, W3, Ws1, Ws2, Ws3, weight):
    blk = 1024
    grid = B // blk
    return pl.pallas_call(
        _head_body,
        grid=(grid,),
        in_specs=[
            pl.BlockSpec((blk, D), lambda i: (i, 0)),
            pl.BlockSpec((RELS, blk, D), lambda i: (0, i, 0)),
            pl.BlockSpec((blk, 1), lambda i: (i, 0)),
            pl.BlockSpec((2 * MLPD, MLPD), lambda i: (0, 0)),
            pl.BlockSpec((2 * MLPD, MLPD), lambda i: (0, 0)),
            pl.BlockSpec((2 * MLPD, MLPD), lambda i: (0, 0)),
            pl.BlockSpec((MLPD, 2), lambda i: (0, 0)),
            pl.BlockSpec((MLPD, 2), lambda i: (0, 0)),
            pl.BlockSpec((MLPD, 2), lambda i: (0, 0)),
            pl.BlockSpec((MLPD + 3 * MLPD, MLPD), lambda i: (0, 0)),
        ],
        out_specs=(pl.BlockSpec((blk, MLPD), lambda i: (i, 0)),
                   pl.BlockSpec((1, 1), lambda i: (0, 0))),
        out_shape=(jax.ShapeDtypeStruct((B, MLPD), jnp.float32),
                   jax.ShapeDtypeStruct((1, 1), jnp.float32)),
    )(f1b, br, lab2d, W1, W2, W3, Ws1, Ws2, Ws3, weight)


def kernel(features, nodes, labels, edge_index1, edge_index2, edge_index3,
           W_mlp, b_mlp, W1, W2, W3, Ws1, Ws2, Ws3, weight):
    nodes = nodes.astype(jnp.int32)
    f1aug = _mlp(features, W_mlp, b_mlp.reshape(1, MLPD))
    inv = jnp.full((N,), -1, jnp.int32).at[nodes].set(
        jnp.arange(B, dtype=jnp.int32))
    pb = inv[nodes]
    invp1 = inv + 1
    inv_packed = invp1[0::2] | (invp1[1::2] << 16)
    parts = _sc_agg(f1aug, inv_packed,
                    edge_index1.astype(jnp.int32),
                    edge_index2.astype(jnp.int32),
                    edge_index3.astype(jnp.int32))
    br, f1b = _sc_batch(parts, pb, nodes, f1aug)
    comb, loss = _tc_head(f1b, br, labels.reshape(B, 1).astype(jnp.int32),
                          W1, W2, W3, Ws1, Ws2, Ws3, weight)
    return comb.T, f1b[:, :MLPD].T, loss.reshape(())


# speedup vs baseline: 4.6512x; 1.0505x over previous
import functools

import jax
import jax.numpy as jnp
from jax import lax
from jax.experimental import pallas as pl
from jax.experimental.pallas import tpu as pltpu
from jax.experimental.pallas import tpu_sc as plsc

N = 50000
FEAT = 128
MLPD = 64
B = 4096
E = 800000

D = 128
NC = 2
NS = 16
NW = NC * NS
C = 1280
NCHUNK = E // C
K = 128
BP = B + 128
ROWS_PER_TILE = BP // NS
RELS = 3


def _mlp_body(x_ref, w_ref, b_ref, out_ref):
    y = jnp.dot(x_ref[...], w_ref[...], preferred_element_type=jnp.float32,
                precision=lax.Precision.HIGHEST)
    y = jnp.maximum(y + b_ref[...], 0.0)
    rows = y.shape[0]
    ones = jnp.ones((rows, 1), jnp.float32)
    pad = jnp.zeros((rows, D - MLPD - 1), jnp.float32)
    out_ref[...] = jnp.concatenate([y, ones, pad], axis=1)


def _mlp(features, W_mlp, b2d):
    blk = 2000
    grid = N // blk
    return pl.pallas_call(
        _mlp_body,
        grid=(grid,),
        in_specs=[
            pl.BlockSpec((blk, FEAT), lambda i: (i, 0)),
            pl.BlockSpec((FEAT, MLPD), lambda i: (0, 0)),
            pl.BlockSpec((1, MLPD), lambda i: (0, 0)),
        ],
        out_specs=pl.BlockSpec((blk, D), lambda i: (i, 0)),
        out_shape=jax.ShapeDtypeStruct((N, D), jnp.float32),
    )(features, W_mlp, b2d)


ZROWS = ROWS_PER_TILE // 3


def _sc_agg_body(f1aug, inv_hbm, e1, e2, e3, out,
                 inv_v, dst_v, src_v, pend_p, pend_s, pidx, sidx, rowbuf,
                 zbuf, sd, ss, sg, ssc, acc):
    c = lax.axis_index("c")
    s = lax.axis_index("s")
    wid = c * NS + s
    edges = [e1, e2, e3]
    base_z = s * ROWS_PER_TILE
    nmy = (NCHUNK - 1 - wid) // NW + 1

    def _zrow(i, _):
        def _zcol(j, __):
            zbuf[i, pl.ds(j * 16, 16)] = jnp.zeros((16,), jnp.float32)
            return 0
        return lax.fori_loop(0, D // 16, _zcol, 0)
    lax.fori_loop(0, ZROWS, _zrow, 0)

    pltpu.sync_copy(inv_hbm, inv_v)

    trash16 = jnp.full((16,), B, jnp.int32)
    zero16 = jnp.zeros((16,), jnp.int32)

    def fire_edges(e, t, ph):
        base = (wid + t * NW) * C
        pltpu.async_copy(e.at[1, pl.ds(base, C)], dst_v[ph], sd[ph])
        pltpu.async_copy(e.at[0, pl.ds(base, C)], src_v[ph], ss[ph])

    def wait_edges(e, ph):
        pltpu.make_async_copy(e.at[1, pl.ds(0, C)], dst_v[ph], sd[ph]).wait()
        pltpu.make_async_copy(e.at[0, pl.ds(0, C)], src_v[ph], ss[ph]).wait()

    def drain_flush(acc, f):
        ph = f & 1
        pltpu.make_async_copy(f1aug.at[sidx.at[ph]], rowbuf.at[ph],
                              sg.at[ph]).wait()
        pltpu.async_copy(rowbuf.at[ph], acc.at[pidx.at[ph]], ssc.at[ph],
                         add=True)

    def wait_scatter(f):
        ph = f & 1
        pltpu.make_async_copy(rowbuf.at[ph], acc.at[pidx.at[ph]],
                              ssc.at[ph]).wait()

    def fire_flush(f, off):
        ph = f & 1

        @pl.when(f >= 2)
        def _():
            wait_scatter(f - 2)

        def cpy(j, ___):
            pidx[ph, pl.ds(j * 16, 16)] = pend_p[pl.ds(off + j * 16, 16)]
            sidx[ph, pl.ds(j * 16, 16)] = pend_s[pl.ds(off + j * 16, 16)]
            return 0
        lax.fori_loop(0, K // 16, cpy, 0)
        pltpu.async_copy(f1aug.at[sidx.at[ph]], rowbuf.at[ph], sg.at[ph])

    def filter_chunk(acc, ph, rem, fc):
        dv, sv = dst_v[ph], src_v[ph]

        def vbody(j, cnt):
            dvec = dv[pl.ds(j * 16, 16)]
            word = plsc.load_gather(inv_v, [lax.shift_right_logical(dvec, 1)])
            sh = lax.shift_left(dvec & 1, 4)
            v = lax.shift_right_logical(word, sh) & 0xFFFF
            p = v - 1
            m = v > 0
            svec = sv[pl.ds(j * 16, 16)]
            plsc.store_compressed(pend_p.at[pl.ds(cnt, 16)], p, mask=m)
            plsc.store_compressed(pend_s.at[pl.ds(cnt, 16)], svec, mask=m)
            return cnt + jnp.sum(m.astype(jnp.int32))
        cnt = lax.fori_loop(0, C // 16, vbody, rem)

        nflush = cnt // K

        def fbody(f, fc2):
            @pl.when(fc2 > 0)
            def _():
                drain_flush(acc, fc2 - 1)
            fire_flush(fc2, f * K)
            return fc2 + 1
        fc = lax.fori_loop(0, nflush, fbody, fc)

        @pl.when(nflush > 0)
        def _():
            def mv(j, __):
                pend_p[pl.ds(j * 16, 16)] = \
                    pend_p[pl.ds(nflush * K + j * 16, 16)]
                pend_s[pl.ds(j * 16, 16)] = \
                    pend_s[pl.ds(nflush * K + j * 16, 16)]
                return 0
            lax.fori_loop(0, K // 16, mv, 0)
        return cnt - nflush * K, fc

    def flush_tail(acc, rem, fc):
        def pad(j, __):
            pend_p[pl.ds(rem + j * 16, 16)] = trash16
            pend_s[pl.ds(rem + j * 16, 16)] = zero16
            return 0
        lax.fori_loop(0, K // 16, pad, 0)

        def last_fire(fc2):
            @pl.when(fc2 > 0)
            def _():
                drain_flush(acc, fc2 - 1)
            fire_flush(fc2, 0)
            return fc2 + 1
        fc = lax.cond(rem > 0, last_fire, lambda fc2: fc2, fc)

        @pl.when(fc > 0)
        def _():
            drain_flush(acc, fc - 1)

        @pl.when(fc >= 2)
        def _():
            wait_scatter(fc - 2)

        @pl.when(fc >= 1)
        def _():
            wait_scatter(fc - 1)

    for r in range(RELS):
        e = edges[r]

        for z in range(3):
            pltpu.sync_copy(zbuf, acc.at[pl.ds(base_z + z * ZROWS, ZROWS)])
        plsc.subcore_barrier()

        fire_edges(e, 0, 0)

        def chunk_step(t, ph, st, e=e, acc=acc):
            wait_edges(e, ph)

            @pl.when(t + 1 < nmy)
            def _():
                fire_edges(e, t + 1, 1 - ph)
            return filter_chunk(acc, ph, st[0], st[1])

        def pair_body(u, st, e=e, acc=acc):
            st = chunk_step(2 * u, 0, st)
            return lax.cond(2 * u + 1 < nmy,
                            lambda s: chunk_step(2 * u + 1, 1, s),
                            lambda s: s, st)

        rem, fc = lax.fori_loop(0, (nmy + 1) // 2, pair_body, (0, 0))
        flush_tail(acc, rem, fc)

        plsc.subcore_barrier()
        pltpu.sync_copy(
            acc.at[pl.ds(base_z, ROWS_PER_TILE)],
            out.at[c * RELS + r, pl.ds(base_z, ROWS_PER_TILE)])
        plsc.subcore_barrier()


def _sc_agg(f1aug, inv, e1, e2, e3):
    mesh = plsc.VectorSubcoreMesh(core_axis_name="c", subcore_axis_name="s")
    fn = functools.partial(
        pl.kernel,
        out_type=jax.ShapeDtypeStruct((NC * RELS, BP, D), jnp.float32),
        mesh=mesh,
        compiler_params=pltpu.CompilerParams(needs_layout_passes=False),
        scratch_types=[
            pltpu.VMEM((N // 2,), jnp.int32),
            (pltpu.VMEM((C,), jnp.int32), pltpu.VMEM((C,), jnp.int32)),
            (pltpu.VMEM((C,), jnp.int32), pltpu.VMEM((C,), jnp.int32)),
            pltpu.VMEM((C + K + 16,), jnp.int32),
            pltpu.VMEM((C + K + 16,), jnp.int32),
            pltpu.VMEM((2, K), jnp.int32),
            pltpu.VMEM((2, K), jnp.int32),
            pltpu.VMEM((2, K, D), jnp.float32),
            pltpu.VMEM((ZROWS, D), jnp.float32),
            (pltpu.SemaphoreType.DMA, pltpu.SemaphoreType.DMA),
            (pltpu.SemaphoreType.DMA, pltpu.SemaphoreType.DMA),
            pltpu.SemaphoreType.DMA((2,)),
            pltpu.SemaphoreType.DMA((2,)),
            pltpu.VMEM_SHARED((BP, D), jnp.float32),
        ],
    )(_sc_agg_body)
    return fn(f1aug, inv, e1, e2, e3)


def _sc_batch_body(parts, pb, nodes, f1aug, br_out, f1b_out,
                   idxv, nidx, buf, sem):
    c = lax.axis_index("c")
    s = lax.axis_index("s")
    wid = c * NS + s
    nb = B // NW
    base = wid * nb

    pltpu.sync_copy(pb.at[pl.ds(base, nb)], idxv)
    pltpu.sync_copy(nodes.at[pl.ds(base, nb)], nidx)

    pltpu.async_copy(f1aug.at[nidx], buf, sem).wait()
    pltpu.sync_copy(buf, f1b_out.at[pl.ds(base, nb)])

    for r in range(RELS):
        pltpu.async_copy(parts.at[r].at[idxv], buf, sem).wait()
        pltpu.async_copy(parts.at[RELS + r].at[idxv], buf, sem, add=True).wait()
        pltpu.sync_copy(buf, br_out.at[r, pl.ds(base, nb)])


def _sc_batch(parts, pb, nodes, f1aug):
    mesh = plsc.VectorSubcoreMesh(core_axis_name="c", subcore_axis_name="s")
    fn = functools.partial(
        pl.kernel,
        out_type=(jax.ShapeDtypeStruct((RELS, B, D), jnp.float32),
                  jax.ShapeDtypeStruct((B, D), jnp.float32)),
        mesh=mesh,
        compiler_params=pltpu.CompilerParams(needs_layout_passes=False),
        scratch_types=[
            pltpu.VMEM((B // NW,), jnp.int32),
            pltpu.VMEM((B // NW,), jnp.int32),
            pltpu.VMEM((B // NW, D), jnp.float32),
            pltpu.SemaphoreType.DMA,
        ],
    )(_sc_batch_body)
    return fn(parts, pb, nodes, f1aug)


def _head_body(f1b_ref, br_ref, lab_ref, w1, w2, w3, ws1, ws2, ws3, wt,
               comb_ref, loss_ref):
    i = pl.program_id(0)
    f1 = f1b_ref[:, :MLPD]
    lab = lab_ref[...]
    hs = [f1]
    loss = jnp.zeros((1, 1), jnp.float32)
    for r, (w, ws) in enumerate(((w1, ws1), (w2, ws2), (w3, ws3))):
        row = br_ref[r]
        ssum = row[:, :MLPD]
        deg = row[:, MLPD:MLPD + 1]
        neigh = ssum / jnp.maximum(deg, 1.0)
        cat = jnp.concatenate([f1, neigh], axis=1)
        h = jnp.maximum(
            jnp.dot(cat, w[...], preferred_element_type=jnp.float32,
                    precision=lax.Precision.HIGHEST), 0.0)
        hs.append(h)
        logits = jnp.dot(h, ws[...], preferred_element_type=jnp.float32,
                         precision=lax.Precision.HIGHEST)
        l0 = logits[:, 0:1]
        l1 = logits[:, 1:2]
        m = jnp.maximum(l0, l1)
        lse = m + jnp.log(jnp.exp(l0 - m) + jnp.exp(l1 - m))
        ll = jnp.where(lab == 0, l0, l1)
        loss = loss + jnp.sum(lse - ll, keepdims=True).reshape(1, 1) / B
    cat2 = jnp.concatenate(hs, axis=1)
    comb_ref[...] = jnp.maximum(
        jnp.dot(cat2, wt[...], preferred_element_type=jnp.float32,
                precision=lax.Precision.HIGHEST), 0.0)

    @pl.when(i == 0)
    def _():
        loss_ref[...] = jnp.zeros((1, 1), jnp.float32)
    loss_ref[...] += loss


def _tc_head(f1b, br, lab2d, W1, W2, W3, Ws1, Ws2, Ws3, weight):
    blk = 1024
    grid = B // blk
    return pl.pallas_call(
        _head_body,
        grid=(grid,),
        in_specs=[
            pl.BlockSpec((blk, D), lambda i: (i, 0)),
            pl.BlockSpec((RELS, blk, D), lambda i: (0, i, 0)),
            pl.BlockSpec((blk, 1), lambda i: (i, 0)),
            pl.BlockSpec((2 * MLPD, MLPD), lambda i: (0, 0)),
            pl.BlockSpec((2 * MLPD, MLPD), lambda i: (0, 0)),
            pl.BlockSpec((2 * MLPD, MLPD), lambda i: (0, 0)),
            pl.BlockSpec((MLPD, 2), lambda i: (0, 0)),
            pl.BlockSpec((MLPD, 2), lambda i: (0, 0)),
            pl.BlockSpec((MLPD, 2), lambda i: (0, 0)),
            pl.BlockSpec((MLPD + 3 * MLPD, MLPD), lambda i: (0, 0)),
        ],
        out_specs=(pl.BlockSpec((blk, MLPD), lambda i: (i, 0)),
                   pl.BlockSpec((1, 1), lambda i: (0, 0))),
        out_shape=(jax.ShapeDtypeStruct((B, MLPD), jnp.float32),
                   jax.ShapeDtypeStruct((1, 1), jnp.float32)),
    )(f1b, br, lab2d, W1, W2, W3, Ws1, Ws2, Ws3, weight)


def kernel(features, nodes, labels, edge_index1, edge_index2, edge_index3,
           W_mlp, b_mlp, W1, W2, W3, Ws1, Ws2, Ws3, weight):
    nodes = nodes.astype(jnp.int32)
    f1aug = _mlp(features, W_mlp, b_mlp.reshape(1, MLPD))
    inv = jnp.full((N,), -1, jnp.int32).at[nodes].set(
        jnp.arange(B, dtype=jnp.int32))
    pb = inv[nodes]
    invp1 = inv + 1
    inv_packed = invp1[0::2] | (invp1[1::2] << 16)
    parts = _sc_agg(f1aug, inv_packed,
                    edge_index1.astype(jnp.int32),
                    edge_index2.astype(jnp.int32),
                    edge_index3.astype(jnp.int32))
    br, f1b = _sc_batch(parts, pb, nodes, f1aug)
    comb, loss = _tc_head(f1b, br, labels.reshape(B, 1).astype(jnp.int32),
                          W1, W2, W3, Ws1, Ws2, Ws3, weight)
    return comb.T, f1b[:, :MLPD].T, loss.reshape(())


# speedup vs baseline: 4.6826x; 1.0068x over previous
import functools

import jax
import jax.numpy as jnp
from jax import lax
from jax.experimental import pallas as pl
from jax.experimental.pallas import tpu as pltpu
from jax.experimental.pallas import tpu_sc as plsc

N = 50000
FEAT = 128
MLPD = 64
B = 4096
E = 800000

D = 128
NC = 2
NS = 16
NW = NC * NS
C = 1280
NCHUNK = E // C
K = 128
BP = B + 128
ROWS_PER_TILE = BP // NS
RELS = 3


def _mlp_body(x_ref, w_ref, b_ref, out_ref):
    y = jnp.dot(x_ref[...], w_ref[...], preferred_element_type=jnp.float32,
                precision=lax.Precision.HIGHEST)
    y = jnp.maximum(y + b_ref[...], 0.0)
    rows = y.shape[0]
    ones = jnp.ones((rows, 1), jnp.float32)
    pad = jnp.zeros((rows, D - MLPD - 1), jnp.float32)
    out_ref[...] = jnp.concatenate([y, ones, pad], axis=1)


def _mlp(features, W_mlp, b2d):
    blk = 2000
    grid = N // blk
    return pl.pallas_call(
        _mlp_body,
        grid=(grid,),
        in_specs=[
            pl.BlockSpec((blk, FEAT), lambda i: (i, 0)),
            pl.BlockSpec((FEAT, MLPD), lambda i: (0, 0)),
            pl.BlockSpec((1, MLPD), lambda i: (0, 0)),
        ],
        out_specs=pl.BlockSpec((blk, D), lambda i: (i, 0)),
        out_shape=jax.ShapeDtypeStruct((N, D), jnp.float32),
    )(features, W_mlp, b2d)


ZROWS = ROWS_PER_TILE // 3


def _sc_agg_body(f1aug, inv_hbm, e1, e2, e3, out,
                 inv_v, dst_v, src_v, pend_p, pend_s, pidx, sidx, rowbuf,
                 zbuf, sd, ss, sg, ssc, acc):
    c = lax.axis_index("c")
    s = lax.axis_index("s")
    wid = c * NS + s
    edges = [e1, e2, e3]
    base_z = s * ROWS_PER_TILE
    nmy = (NCHUNK - 1 - wid) // NW + 1

    def _zrow(i, _):
        def _zcol(j, __):
            zbuf[i, pl.ds(j * 16, 16)] = jnp.zeros((16,), jnp.float32)
            return 0
        return lax.fori_loop(0, D // 16, _zcol, 0)
    lax.fori_loop(0, ZROWS, _zrow, 0)

    pltpu.sync_copy(inv_hbm, inv_v)

    trash16 = jnp.full((16,), B, jnp.int32)
    zero16 = jnp.zeros((16,), jnp.int32)

    def fire_edges(e, t, ph):
        base = (wid + t * NW) * C
        pltpu.async_copy(e.at[1, pl.ds(base, C)], dst_v[ph], sd[ph])
        pltpu.async_copy(e.at[0, pl.ds(base, C)], src_v[ph], ss[ph])

    def wait_edges(e, ph):
        pltpu.make_async_copy(e.at[1, pl.ds(0, C)], dst_v[ph], sd[ph]).wait()
        pltpu.make_async_copy(e.at[0, pl.ds(0, C)], src_v[ph], ss[ph]).wait()

    def drain_flush(acc, f):
        ph = f % 3
        pltpu.make_async_copy(f1aug.at[sidx.at[ph]], rowbuf.at[ph],
                              sg.at[ph]).wait()
        pltpu.async_copy(rowbuf.at[ph], acc.at[pidx.at[ph]], ssc.at[ph],
                         add=True)

    def wait_scatter(f):
        ph = f % 3
        pltpu.make_async_copy(rowbuf.at[ph], acc.at[pidx.at[ph]],
                              ssc.at[ph]).wait()

    def fire_flush(f, off):
        ph = f % 3

        @pl.when(f >= 3)
        def _():
            wait_scatter(f - 3)

        def cpy(j, ___):
            pidx[ph, pl.ds(j * 16, 16)] = pend_p[pl.ds(off + j * 16, 16)]
            sidx[ph, pl.ds(j * 16, 16)] = pend_s[pl.ds(off + j * 16, 16)]
            return 0
        lax.fori_loop(0, K // 16, cpy, 0)
        pltpu.async_copy(f1aug.at[sidx.at[ph]], rowbuf.at[ph], sg.at[ph])

    def filter_chunk(acc, ph, rem, fc):
        dv, sv = dst_v[ph], src_v[ph]

        def vbody(j, cnt):
            dvec = dv[pl.ds(j * 16, 16)]
            word = plsc.load_gather(inv_v, [lax.shift_right_logical(dvec, 1)])
            sh = lax.shift_left(dvec & 1, 4)
            v = lax.shift_right_logical(word, sh) & 0xFFFF
            p = v - 1
            m = v > 0
            svec = sv[pl.ds(j * 16, 16)]
            plsc.store_compressed(pend_p.at[pl.ds(cnt, 16)], p, mask=m)
            plsc.store_compressed(pend_s.at[pl.ds(cnt, 16)], svec, mask=m)
            return cnt + jnp.sum(m.astype(jnp.int32))
        cnt = lax.fori_loop(0, C // 16, vbody, rem)

        nflush = cnt // K

        def fbody(f, fc2):
            @pl.when(fc2 >= 2)
            def _():
                drain_flush(acc, fc2 - 2)
            fire_flush(fc2, f * K)
            return fc2 + 1
        fc = lax.fori_loop(0, nflush, fbody, fc)

        @pl.when(nflush > 0)
        def _():
            def mv(j, __):
                pend_p[pl.ds(j * 16, 16)] = \
                    pend_p[pl.ds(nflush * K + j * 16, 16)]
                pend_s[pl.ds(j * 16, 16)] = \
                    pend_s[pl.ds(nflush * K + j * 16, 16)]
                return 0
            lax.fori_loop(0, K // 16, mv, 0)
        return cnt - nflush * K, fc

    def flush_tail(acc, rem, fc):
        def pad(j, __):
            pend_p[pl.ds(rem + j * 16, 16)] = trash16
            pend_s[pl.ds(rem + j * 16, 16)] = zero16
            return 0
        lax.fori_loop(0, K // 16, pad, 0)

        def last_fire(fc2):
            @pl.when(fc2 >= 2)
            def _():
                drain_flush(acc, fc2 - 2)
            fire_flush(fc2, 0)
            return fc2 + 1
        fc = lax.cond(rem > 0, last_fire, lambda fc2: fc2, fc)

        @pl.when(fc >= 2)
        def _():
            drain_flush(acc, fc - 2)

        @pl.when(fc >= 1)
        def _():
            drain_flush(acc, fc - 1)

        @pl.when(fc >= 3)
        def _():
            wait_scatter(fc - 3)

        @pl.when(fc >= 2)
        def _():
            wait_scatter(fc - 2)

        @pl.when(fc >= 1)
        def _():
            wait_scatter(fc - 1)

    for r in range(RELS):
        e = edges[r]

        for z in range(3):
            pltpu.sync_copy(zbuf, acc.at[pl.ds(base_z + z * ZROWS, ZROWS)])
        plsc.subcore_barrier()

        fire_edges(e, 0, 0)

        def chunk_step(t, ph, st, e=e, acc=acc):
            wait_edges(e, ph)

            @pl.when(t + 1 < nmy)
            def _():
                fire_edges(e, t + 1, 1 - ph)
            return filter_chunk(acc, ph, st[0], st[1])

        def pair_body(u, st, e=e, acc=acc):
            st = chunk_step(2 * u, 0, st)
            return lax.cond(2 * u + 1 < nmy,
                            lambda s: chunk_step(2 * u + 1, 1, s),
                            lambda s: s, st)

        rem, fc = lax.fori_loop(0, (nmy + 1) // 2, pair_body, (0, 0))
        flush_tail(acc, rem, fc)

        plsc.subcore_barrier()
        pltpu.sync_copy(
            acc.at[pl.ds(base_z, ROWS_PER_TILE)],
            out.at[c * RELS + r, pl.ds(base_z, ROWS_PER_TILE)])
        plsc.subcore_barrier()


def _sc_agg(f1aug, inv, e1, e2, e3):
    mesh = plsc.VectorSubcoreMesh(core_axis_name="c", subcore_axis_name="s")
    fn = functools.partial(
        pl.kernel,
        out_type=jax.ShapeDtypeStruct((NC * RELS, BP, D), jnp.float32),
        mesh=mesh,
        compiler_params=pltpu.CompilerParams(needs_layout_passes=False),
        scratch_types=[
            pltpu.VMEM((N // 2,), jnp.int32),
            (pltpu.VMEM((C,), jnp.int32), pltpu.VMEM((C,), jnp.int32)),
            (pltpu.VMEM((C,), jnp.int32), pltpu.VMEM((C,), jnp.int32)),
            pltpu.VMEM((C + K + 16,), jnp.int32),
            pltpu.VMEM((C + K + 16,), jnp.int32),
            pltpu.VMEM((3, K), jnp.int32),
            pltpu.VMEM((3, K), jnp.int32),
            pltpu.VMEM((3, K, D), jnp.float32),
            pltpu.VMEM((ZROWS, D), jnp.float32),
            (pltpu.SemaphoreType.DMA, pltpu.SemaphoreType.DMA),
            (pltpu.SemaphoreType.DMA, pltpu.SemaphoreType.DMA),
            pltpu.SemaphoreType.DMA((3,)),
            pltpu.SemaphoreType.DMA((3,)),
            pltpu.VMEM_SHARED((BP, D), jnp.float32),
        ],
    )(_sc_agg_body)
    return fn(f1aug, inv, e1, e2, e3)


def _sc_batch_body(parts, pb, nodes, f1aug, br_out, f1b_out,
                   idxv, nidx, buf, sem):
    c = lax.axis_index("c")
    s = lax.axis_index("s")
    wid = c * NS + s
    nb = B // NW
    base = wid * nb

    pltpu.sync_copy(pb.at[pl.ds(base, nb)], idxv)
    pltpu.sync_copy(nodes.at[pl.ds(base, nb)], nidx)

    pltpu.async_copy(f1aug.at[nidx], buf, sem).wait()
    pltpu.sync_copy(buf, f1b_out.at[pl.ds(base, nb)])

    for r in range(RELS):
        pltpu.async_copy(parts.at[r].at[idxv], buf, sem).wait()
        pltpu.async_copy(parts.at[RELS + r].at[idxv], buf, sem, add=True).wait()
        pltpu.sync_copy(buf, br_out.at[r, pl.ds(base, nb)])


def _sc_batch(parts, pb, nodes, f1aug):
    mesh = plsc.VectorSubcoreMesh(core_axis_name="c", subcore_axis_name="s")
    fn = functools.partial(
        pl.kernel,
        out_type=(jax.ShapeDtypeStruct((RELS, B, D), jnp.float32),
                  jax.ShapeDtypeStruct((B, D), jnp.float32)),
        mesh=mesh,
        compiler_params=pltpu.CompilerParams(needs_layout_passes=False),
        scratch_types=[
            pltpu.VMEM((B // NW,), jnp.int32),
            pltpu.VMEM((B // NW,), jnp.int32),
            pltpu.VMEM((B // NW, D), jnp.float32),
            pltpu.SemaphoreType.DMA,
        ],
    )(_sc_batch_body)
    return fn(parts, pb, nodes, f1aug)


def _head_body(f1b_ref, br_ref, lab_ref, w1, w2, w3, ws1, ws2, ws3, wt,
               comb_ref, loss_ref):
    i = pl.program_id(0)
    f1 = f1b_ref[:, :MLPD]
    lab = lab_ref[...]
    hs = [f1]
    loss = jnp.zeros((1, 1), jnp.float32)
    for r, (w, ws) in enumerate(((w1, ws1), (w2, ws2), (w3, ws3))):
        row = br_ref[r]
        ssum = row[:, :MLPD]
        deg = row[:, MLPD:MLPD + 1]
        neigh = ssum / jnp.maximum(deg, 1.0)
        cat = jnp.concatenate([f1, neigh], axis=1)
        h = jnp.maximum(
            jnp.dot(cat, w[...], preferred_element_type=jnp.float32,
                    precision=lax.Precision.HIGHEST), 0.0)
        hs.append(h)
        logits = jnp.dot(h, ws[...], preferred_element_type=jnp.float32,
                         precision=lax.Precision.HIGHEST)
        l0 = logits[:, 0:1]
        l1 = logits[:, 1:2]
        m = jnp.maximum(l0, l1)
        lse = m + jnp.log(jnp.exp(l0 - m) + jnp.exp(l1 - m))
        ll = jnp.where(lab == 0, l0, l1)
        loss = loss + jnp.sum(lse - ll, keepdims=True).reshape(1, 1) / B
    cat2 = jnp.concatenate(hs, axis=1)
    comb_ref[...] = jnp.maximum(
        jnp.dot(cat2, wt[...], preferred_element_type=jnp.float32,
                precision=lax.Precision.HIGHEST), 0.0)

    @pl.when(i == 0)
    def _():
        loss_ref[...] = jnp.zeros((1, 1), jnp.float32)
    loss_ref[...] += loss


def _tc_head(f1b, br, lab2d, W1, W2, W3, Ws1, Ws2, Ws3, weight):
    blk = 1024
    grid = B // blk
    return pl.pallas_call(
        _head_body,
        grid=(grid,),
        in_specs=[
            pl.BlockSpec((blk, D), lambda i: (i, 0)),
            pl.BlockSpec((RELS, blk, D), lambda i: (0, i, 0)),
            pl.BlockSpec((blk, 1), lambda i: (i, 0)),
            pl.BlockSpec((2 * MLPD, MLPD), lambda i: (0, 0)),
            pl.BlockSpec((2 * MLPD, MLPD), lambda i: (0, 0)),
            pl.BlockSpec((2 * MLPD, MLPD), lambda i: (0, 0)),
            pl.BlockSpec((MLPD, 2), lambda i: (0, 0)),
            pl.BlockSpec((MLPD, 2), lambda i: (0, 0)),
            pl.BlockSpec((MLPD, 2), lambda i: (0, 0)),
            pl.BlockSpec((MLPD + 3 * MLPD, MLPD), lambda i: (0, 0)),
        ],
        out_specs=(pl.BlockSpec((blk, MLPD), lambda i: (i, 0)),
                   pl.BlockSpec((1, 1), lambda i: (0, 0))),
        out_shape=(jax.ShapeDtypeStruct((B, MLPD), jnp.float32),
                   jax.ShapeDtypeStruct((1, 1), jnp.float32)),
    )(f1b, br, lab2d, W1, W2, W3, Ws1, Ws2, Ws3, weight)


def kernel(features, nodes, labels, edge_index1, edge_index2, edge_index3,
           W_mlp, b_mlp, W1, W2, W3, Ws1, Ws2, Ws3, weight):
    nodes = nodes.astype(jnp.int32)
    f1aug = _mlp(features, W_mlp, b_mlp.reshape(1, MLPD))
    inv = jnp.full((N,), -1, jnp.int32).at[nodes].set(
        jnp.arange(B, dtype=jnp.int32))
    pb = inv[nodes]
    invp1 = inv + 1
    inv_packed = invp1[0::2] | (invp1[1::2] << 16)
    parts = _sc_agg(f1aug, inv_packed,
                    edge_index1.astype(jnp.int32),
                    edge_index2.astype(jnp.int32),
                    edge_index3.astype(jnp.int32))
    br, f1b = _sc_batch(parts, pb, nodes, f1aug)
    comb, loss = _tc_head(f1b, br, labels.reshape(B, 1).astype(jnp.int32),
                          W1, W2, W3, Ws1, Ws2, Ws3, weight)
    return comb.T, f1b[:, :MLPD].T, loss.reshape(())
